# baseline (speedup 1.0000x reference)
import functools

import jax
import jax.numpy as jnp
from jax import lax
from jax.experimental import pallas as pl
from jax.experimental.pallas import tpu as pltpu
from jax.experimental.pallas import tpu_sc as plsc

N = 10000
E = 320000
D = 128
C = 40

NC, NS = 2, 16
NW = NC * NS
E_PER_W = E // NW
CHUNK = 80
NCH = E_PER_W // CHUNK
NP = 10240
RPS = NP // NS
PH = 64

BLK = 2000
CBLK = 80

_SC_MESH = plsc.VectorSubcoreMesh(core_axis_name="c", subcore_axis_name="s")


@functools.partial(
    pl.kernel,
    out_type=jax.ShapeDtypeStruct((NC * NP, D), jnp.float32),
    mesh=_SC_MESH,
    scratch_types=[
        pltpu.VMEM((PH, CHUNK), jnp.int32),
        pltpu.VMEM((PH, CHUNK), jnp.int32),
        pltpu.VMEM((CHUNK, D), jnp.float32),
        pltpu.VMEM((CHUNK, D), jnp.float32),
        pltpu.VMEM_SHARED((NP, D), jnp.float32),
        pltpu.SemaphoreType.DMA,
        pltpu.SemaphoreType.DMA,
        pltpu.SemaphoreType.DMA,
        pltpu.SemaphoreType.DMA,
    ],
)
def _sc_aggregate(m_hbm, edges_hbm, z_hbm, out_hbm,
                  sidx, didx, buf0, buf1, acc, sem0a, sem0b, sem1a, sem1b):
    cid = lax.axis_index("c")
    sid = lax.axis_index("s")
    wid = cid * NS + sid

    pltpu.sync_copy(z_hbm, acc.at[pl.ds(sid * RPS, RPS)])
    plsc.subcore_barrier()

    HC = CHUNK // 2

    def gather2(j, buf, sa, sb):
        pltpu.async_copy(m_hbm.at[sidx.at[j, pl.ds(0, HC)]],
                         buf.at[pl.ds(0, HC)], sa)
        pltpu.async_copy(m_hbm.at[sidx.at[j, pl.ds(HC, HC)]],
                         buf.at[pl.ds(HC, HC)], sb)

    def wait2(buf, sa, sb):
        pltpu.make_async_copy(m_hbm.at[sidx.at[0, pl.ds(0, HC)]],
                              buf.at[pl.ds(0, HC)], sa).wait()
        pltpu.make_async_copy(m_hbm.at[sidx.at[0, pl.ds(HC, HC)]],
                              buf.at[pl.ds(HC, HC)], sb).wait()

    def run_phase(base, count):
        pltpu.sync_copy(edges_hbm.at[0, wid, pl.ds(base, count)],
                        sidx.at[pl.ds(0, count)])
        pltpu.sync_copy(edges_hbm.at[1, wid, pl.ds(base, count)],
                        didx.at[pl.ds(0, count)])
        gather2(0, buf0, sem0a, sem0b)

        @pl.loop(0, count - (count % 2), step=2)
        def _(j):
            gather2(j + 1, buf1, sem1a, sem1b)
            wait2(buf0, sem0a, sem0b)
            pltpu.sync_copy(buf0, acc.at[didx.at[j]], add=True)

            @pl.when(j + 2 < count)
            def _():
                gather2(j + 2, buf0, sem0a, sem0b)

            wait2(buf1, sem1a, sem1b)
            pltpu.sync_copy(buf1, acc.at[didx.at[j + 1]], add=True)

        if count % 2:
            wait2(buf0, sem0a, sem0b)
            pltpu.sync_copy(buf0, acc.at[didx.at[count - 1]], add=True)

    run_phase(0, PH)
    run_phase(PH, NCH - PH)

    plsc.subcore_barrier()
    pltpu.sync_copy(acc.at[pl.ds(sid * RPS, RPS)],
                    out_hbm.at[pl.ds(cid * NP + sid * RPS, RPS)])


def _bmm(a, w):
    return jnp.dot(a.astype(jnp.bfloat16), w.astype(jnp.bfloat16),
                   preferred_element_type=jnp.float32)


def _tc_encode(x, enc_W0, enc_b0, gcn_W0):
    def body(x_ref, w0_ref, b0_ref, w1_ref, o_ref):
        h = _bmm(x_ref[...], w0_ref[...]) + b0_ref[...]
        o_ref[...] = _bmm(h, w1_ref[...])

    return pl.pallas_call(
        body,
        grid=(N // BLK,),
        in_specs=[
            pl.BlockSpec((BLK, D), lambda i: (i, 0)),
            pl.BlockSpec((D, D), lambda i: (0, 0)),
            pl.BlockSpec((1, D), lambda i: (0, 0)),
            pl.BlockSpec((D, D), lambda i: (0, 0)),
        ],
        out_specs=pl.BlockSpec((BLK, D), lambda i: (i, 0)),
        out_shape=jax.ShapeDtypeStruct((N, D), jnp.float32),
    )(x, enc_W0, enc_b0, gcn_W0)


def _tc_conv_out(p3, b, W):
    def body(p_ref, b_ref, w_ref, o_ref):
        h = jax.nn.relu(p_ref[0] + p_ref[1] + b_ref[...])
        o_ref[...] = _bmm(h, w_ref[...])

    return pl.pallas_call(
        body,
        grid=(N // BLK,),
        in_specs=[
            pl.BlockSpec((2, BLK, D), lambda i: (0, i, 0)),
            pl.BlockSpec((1, D), lambda i: (0, 0)),
            pl.BlockSpec((D, D), lambda i: (0, 0)),
        ],
        out_specs=pl.BlockSpec((BLK, D), lambda i: (i, 0)),
        out_shape=jax.ShapeDtypeStruct((N, D), jnp.float32),
    )(p3, b, W)


def _tc_decode(q3, b, decW_pad, dec_b_pad):
    def body(q_ref, b_ref, w_ref, db_ref, o_ref):
        h = jax.nn.relu(q_ref[0] + q_ref[1] + b_ref[...])
        logits = _bmm(h, w_ref[...]) + db_ref[...]
        mx = jnp.max(logits, axis=1, keepdims=True)
        lse = jnp.log(jnp.sum(jnp.exp(logits - mx), axis=1, keepdims=True))
        o_ref[...] = (logits - mx - lse)[:, :C]

    return pl.pallas_call(
        body,
        grid=(N // BLK,),
        in_specs=[
            pl.BlockSpec((2, BLK, D), lambda i: (0, i, 0)),
            pl.BlockSpec((1, D), lambda i: (0, 0)),
            pl.BlockSpec((D, D), lambda i: (0, 0)),
            pl.BlockSpec((1, D), lambda i: (0, 0)),
        ],
        out_specs=pl.BlockSpec((BLK, C), lambda i: (i, 0)),
        out_shape=jax.ShapeDtypeStruct((N, C), jnp.float32),
    )(q3, b, decW_pad, dec_b_pad)


def kernel(x, edge_index, enc_W0, enc_b0, gcn_W0, gcn_b0, gcn_W1, gcn_b1,
           dec_W0, dec_b0):
    edges = edge_index.astype(jnp.int32).reshape(2, NW, NCH, CHUNK)
    zeros = jnp.zeros((RPS, D), jnp.float32)

    decW_pad = jnp.pad(dec_W0, ((0, 0), (0, D - C)))
    dec_b_pad = jnp.pad(dec_b0, (0, D - C), constant_values=-1e30)

    m0 = _tc_encode(x, enc_W0, enc_b0.reshape(1, D), gcn_W0)
    p = _sc_aggregate(m0, edges, zeros).reshape(2, NP, D)
    m1 = _tc_conv_out(p, gcn_b0.reshape(1, D), gcn_W1)
    q = _sc_aggregate(m1, edges, zeros).reshape(2, NP, D)
    return _tc_decode(q, gcn_b1.reshape(1, D), decW_pad,
                      dec_b_pad.reshape(1, D))


# speedup vs baseline: 1.0082x; 1.0082x over previous
import functools

import jax
import jax.numpy as jnp
from jax import lax
from jax.experimental import pallas as pl
from jax.experimental.pallas import tpu as pltpu
from jax.experimental.pallas import tpu_sc as plsc

N = 10000
E = 320000
D = 128
C = 40

NC, NS = 2, 16
NW = NC * NS
E_PER_W = E // NW
CHUNK = 80
NCH = E_PER_W // CHUNK
NP = 10240
RPS = NP // NS
PH = 64

BLK = 2000
CBLK = 80

_SC_MESH = plsc.VectorSubcoreMesh(core_axis_name="c", subcore_axis_name="s")


@functools.partial(
    pl.kernel,
    out_type=jax.ShapeDtypeStruct((NC * NP, D), jnp.float32),
    mesh=_SC_MESH,
    scratch_types=[
        pltpu.VMEM((PH, CHUNK), jnp.int32),
        pltpu.VMEM((PH, CHUNK), jnp.int32),
        pltpu.VMEM((CHUNK, D), jnp.float32),
        pltpu.VMEM((CHUNK, D), jnp.float32),
        pltpu.VMEM_SHARED((NP, D), jnp.float32),
        pltpu.SemaphoreType.DMA,
        pltpu.SemaphoreType.DMA,
        pltpu.SemaphoreType.DMA,
        pltpu.SemaphoreType.DMA,
    ],
)
def _sc_aggregate(m_hbm, edges_hbm, z_hbm, out_hbm,
                  sidx, didx, buf0, buf1, acc, sem0a, sem0b, sem1a, sem1b):
    cid = lax.axis_index("c")
    sid = lax.axis_index("s")
    wid = cid * NS + sid

    HC = CHUNK // 2

    def gather2(j, buf, sa, sb):
        pltpu.async_copy(m_hbm.at[sidx.at[j, pl.ds(0, HC)]],
                         buf.at[pl.ds(0, HC)], sa)
        pltpu.async_copy(m_hbm.at[sidx.at[j, pl.ds(HC, HC)]],
                         buf.at[pl.ds(HC, HC)], sb)

    def wait2(buf, sa, sb):
        pltpu.make_async_copy(m_hbm.at[sidx.at[0, pl.ds(0, HC)]],
                              buf.at[pl.ds(0, HC)], sa).wait()
        pltpu.make_async_copy(m_hbm.at[sidx.at[0, pl.ds(HC, HC)]],
                              buf.at[pl.ds(HC, HC)], sb).wait()

    def run_phase(base, count, prologue=True):
        if prologue:
            pltpu.sync_copy(edges_hbm.at[0, wid, pl.ds(base, count)],
                            sidx.at[pl.ds(0, count)])
            pltpu.sync_copy(edges_hbm.at[1, wid, pl.ds(base, count)],
                            didx.at[pl.ds(0, count)])
            gather2(0, buf0, sem0a, sem0b)

        @pl.loop(0, count - (count % 2), step=2)
        def _(j):
            gather2(j + 1, buf1, sem1a, sem1b)
            wait2(buf0, sem0a, sem0b)
            pltpu.sync_copy(buf0, acc.at[didx.at[j]], add=True)

            @pl.when(j + 2 < count)
            def _():
                gather2(j + 2, buf0, sem0a, sem0b)

            wait2(buf1, sem1a, sem1b)
            pltpu.sync_copy(buf1, acc.at[didx.at[j + 1]], add=True)

        if count % 2:
            wait2(buf0, sem0a, sem0b)
            pltpu.sync_copy(buf0, acc.at[didx.at[count - 1]], add=True)

    pltpu.sync_copy(edges_hbm.at[0, wid, pl.ds(0, PH)], sidx)
    pltpu.sync_copy(edges_hbm.at[1, wid, pl.ds(0, PH)], didx)
    gather2(0, buf0, sem0a, sem0b)
    pltpu.sync_copy(z_hbm, acc.at[pl.ds(sid * RPS, RPS)])
    plsc.subcore_barrier()

    run_phase(0, PH, prologue=False)
    run_phase(PH, NCH - PH)

    plsc.subcore_barrier()
    pltpu.sync_copy(acc.at[pl.ds(sid * RPS, RPS)],
                    out_hbm.at[pl.ds(cid * NP + sid * RPS, RPS)])


def _tc_encode(x, enc_W0, enc_b0, gcn_W0):
    def body(x_ref, w0_ref, b0_ref, w1_ref, o_ref):
        h = jnp.dot(x_ref[...], w0_ref[...],
                    preferred_element_type=jnp.float32) + b0_ref[...]
        o_ref[...] = jnp.dot(h, w1_ref[...], preferred_element_type=jnp.float32)

    return pl.pallas_call(
        body,
        grid=(N // BLK,),
        in_specs=[
            pl.BlockSpec((BLK, D), lambda i: (i, 0)),
            pl.BlockSpec((D, D), lambda i: (0, 0)),
            pl.BlockSpec((1, D), lambda i: (0, 0)),
            pl.BlockSpec((D, D), lambda i: (0, 0)),
        ],
        out_specs=pl.BlockSpec((BLK, D), lambda i: (i, 0)),
        out_shape=jax.ShapeDtypeStruct((N, D), jnp.float32),
    )(x, enc_W0, enc_b0, gcn_W0)


def _tc_conv_out(p3, b, W):
    def body(p_ref, b_ref, w_ref, o_ref):
        h = jax.nn.relu(p_ref[0] + p_ref[1] + b_ref[...])
        o_ref[...] = jnp.dot(h, w_ref[...], preferred_element_type=jnp.float32)

    return pl.pallas_call(
        body,
        grid=(N // BLK,),
        in_specs=[
            pl.BlockSpec((2, BLK, D), lambda i: (0, i, 0)),
            pl.BlockSpec((1, D), lambda i: (0, 0)),
            pl.BlockSpec((D, D), lambda i: (0, 0)),
        ],
        out_specs=pl.BlockSpec((BLK, D), lambda i: (i, 0)),
        out_shape=jax.ShapeDtypeStruct((N, D), jnp.float32),
    )(p3, b, W)


def _tc_decode(q3, b, decW_pad, dec_b_pad):
    def body(q_ref, b_ref, w_ref, db_ref, o_ref):
        h = jax.nn.relu(q_ref[0] + q_ref[1] + b_ref[...])
        logits = jnp.dot(h, w_ref[...],
                         preferred_element_type=jnp.float32) + db_ref[...]
        mx = jnp.max(logits, axis=1, keepdims=True)
        lse = jnp.log(jnp.sum(jnp.exp(logits - mx), axis=1, keepdims=True))
        o_ref[...] = (logits - mx - lse)[:, :C]

    return pl.pallas_call(
        body,
        grid=(N // BLK,),
        in_specs=[
            pl.BlockSpec((2, BLK, D), lambda i: (0, i, 0)),
            pl.BlockSpec((1, D), lambda i: (0, 0)),
            pl.BlockSpec((D, D), lambda i: (0, 0)),
            pl.BlockSpec((1, D), lambda i: (0, 0)),
        ],
        out_specs=pl.BlockSpec((BLK, C), lambda i: (i, 0)),
        out_shape=jax.ShapeDtypeStruct((N, C), jnp.float32),
    )(q3, b, decW_pad, dec_b_pad)


def kernel(x, edge_index, enc_W0, enc_b0, gcn_W0, gcn_b0, gcn_W1, gcn_b1,
           dec_W0, dec_b0):
    edges = edge_index.astype(jnp.int32).reshape(2, NW, NCH, CHUNK)
    zeros = jnp.zeros((RPS, D), jnp.float32)

    decW_pad = jnp.pad(dec_W0, ((0, 0), (0, D - C)))
    dec_b_pad = jnp.pad(dec_b0, (0, D - C), constant_values=-1e30)

    m0 = _tc_encode(x, enc_W0, enc_b0.reshape(1, D), gcn_W0)
    p = _sc_aggregate(m0, edges, zeros).reshape(2, NP, D)
    m1 = _tc_conv_out(p, gcn_b0.reshape(1, D), gcn_W1)
    q = _sc_aggregate(m1, edges, zeros).reshape(2, NP, D)
    return _tc_decode(q, gcn_b1.reshape(1, D), decW_pad,
                      dec_b_pad.reshape(1, D))
